# baseline (speedup 1.0000x reference)
import functools

import jax
import jax.numpy as jnp
from jax import lax
from jax.experimental import pallas as pl
from jax.experimental.pallas import tpu as pltpu
from jax.experimental.pallas import tpu_sc as plsc

N = 10000
E = 320000
D = 128
H = 8
HD = 16

NC = 2
NS = 16
NW = NC * NS

CH = 80
EW = E // NW
CHG = 80
NCH_G = EW // CHG
NP_G = NCH_G // 2
E2 = E // 2
ET = E2 // NS
NCH_S = ET // CH
NCH_E = EW // CH

NHALF = N // 2
ACC_ROWS = 5120
NROW_W = ACC_ROWS // NS

NPAD_ES = 10240
ES_ROWS = NPAD_ES * H // D
ES_SLAB = ES_ROWS // NS

_f32 = jnp.float32


def _proj_body(x_ref, wq_ref, wk_ref, wv_ref, bq_ref, nq_ref, nk_ref, nv_ref):
    x = x_ref[...]
    nq_ref[...] = jnp.dot(x, wq_ref[...], preferred_element_type=_f32) + bq_ref[...]
    nk_ref[...] = jnp.dot(x, wk_ref[...], preferred_element_type=_f32)
    nv_ref[...] = jnp.dot(x, wv_ref[...], preferred_element_type=_f32)


BP = 2000


def _node_proj(x, wq, wk, wv, bq):
    nblk = pl.BlockSpec((BP, D), lambda i: (i, 0))
    wblk = pl.BlockSpec((D, D), lambda i: (0, 0))
    vblk = pl.BlockSpec((1, D), lambda i: (0, 0))
    return pl.pallas_call(
        _proj_body,
        grid=(N // BP,),
        in_specs=[nblk, wblk, wblk, wblk, vblk],
        out_specs=[nblk] * 3,
        out_shape=[jax.ShapeDtypeStruct((N, D), _f32)] * 3,
    )(x, wq, wk, wv, bq)


def _gather_body(nk_hbm, nq_hbm, nv_hbm, src2_hbm, dst2_hbm,
                 nks_hbm, nqd_hbm, nvs_hbm,
                 idxs_v, idxd_v, ra0, rb0, rc0, ra1, rb1, rc1,
                 sa0, sb0, sc0, sa1, sb1, sc1):
    wid = lax.axis_index("s") * NC + lax.axis_index("c")
    cb = wid * NCH_G
    pltpu.sync_copy(src2_hbm.at[wid], idxs_v)
    pltpu.sync_copy(dst2_hbm.at[wid], idxd_v)

    def issue(j, ra, rb, rc, sa, sb, sc_):
        da = pltpu.async_copy(nk_hbm.at[idxs_v.at[j]], ra, sa)
        db = pltpu.async_copy(nq_hbm.at[idxd_v.at[j]], rb, sb)
        dc = pltpu.async_copy(nv_hbm.at[idxs_v.at[j]], rc, sc_)
        return da, db, dc

    def store(j, ra, rb, rc):
        eb = (cb + j) * CHG
        pltpu.sync_copy(ra, nks_hbm.at[pl.ds(eb, CHG)])
        pltpu.sync_copy(rb, nqd_hbm.at[pl.ds(eb, CHG)])
        pltpu.sync_copy(rc, nvs_hbm.at[pl.ds(eb, CHG)])

    def wait(j, ra, rb, rc, sa, sb, sc_):
        pltpu.make_async_copy(nk_hbm.at[idxs_v.at[j]], ra, sa).wait()
        pltpu.make_async_copy(nq_hbm.at[idxd_v.at[j]], rb, sb).wait()
        pltpu.make_async_copy(nv_hbm.at[idxs_v.at[j]], rc, sc_).wait()

    issue(0, ra0, rb0, rc0, sa0, sb0, sc0)

    def body(i, carry):
        j = 2 * i
        wait(j, ra0, rb0, rc0, sa0, sb0, sc0)
        issue(j + 1, ra1, rb1, rc1, sa1, sb1, sc1)
        store(j, ra0, rb0, rc0)
        wait(j + 1, ra1, rb1, rc1, sa1, sb1, sc1)
        issue(j + 2, ra0, rb0, rc0, sa0, sb0, sc0)
        store(j + 1, ra1, rb1, rc1)
        return carry

    lax.fori_loop(0, NP_G, body, 0)
    j = NCH_G - 1
    wait(j, ra0, rb0, rc0, sa0, sb0, sc0)
    store(j, ra0, rb0, rc0)


def _sc_gather(nk, nq, nv, src2, dst2):
    mesh = plsc.VectorSubcoreMesh(core_axis_name="c", subcore_axis_name="s")
    rt = pltpu.VMEM((CHG, D), _f32)
    f = functools.partial(
        pl.kernel,
        out_type=[jax.ShapeDtypeStruct((E, D), _f32)] * 3,
        mesh=mesh,
        scratch_types=[
            pltpu.VMEM((NCH_G, CHG), jnp.int32),
            pltpu.VMEM((NCH_G, CHG), jnp.int32),
            rt, rt, rt, rt, rt, rt,
            pltpu.SemaphoreType.DMA,
            pltpu.SemaphoreType.DMA,
            pltpu.SemaphoreType.DMA,
            pltpu.SemaphoreType.DMA,
            pltpu.SemaphoreType.DMA,
            pltpu.SemaphoreType.DMA,
        ],
    )(_gather_body)
    return f(nk, nq, nv, src2, dst2)


BE = 2000


def _edge_body(ea_ref, nks_ref, nqd_ref, nvs_ref, weq_ref, beq_ref,
               weo_ref, beo_ref, awf_ref, gs8_ref, k8_ref, ewbd_ref,
               g1e_ref, b1e_ref,
               es8_ref, w_ref, eh_ref):
    ea = ea_ref[...]
    eq = jnp.dot(ea, weq_ref[...], preferred_element_type=_f32) + beq_ref[...]
    conn = jnp.maximum(nks_ref[...] + nqd_ref[...] + eq, 0.0)
    score8 = jnp.dot(conn * awf_ref[...], gs8_ref[...],
                     preferred_element_type=_f32)
    es8 = jnp.exp(jnp.clip(score8, -5.0, 5.0))
    es8_ref[...] = es8
    es128 = jnp.dot(es8, k8_ref[...], preferred_element_type=_f32)
    w_ref[...] = (nvs_ref[...]
                  + jnp.dot(conn, ewbd_ref[...], preferred_element_type=_f32)
                  ) * es128
    eh = ea + jnp.dot(conn, weo_ref[...], preferred_element_type=_f32) + beo_ref[...]
    m = jnp.mean(eh, axis=-1, keepdims=True)
    v = jnp.mean((eh - m) * (eh - m), axis=-1, keepdims=True)
    eh = (eh - m) * lax.rsqrt(v + 1e-5) * g1e_ref[...] + b1e_ref[...]
    eh_ref[...] = jnp.maximum(eh, 0.0)


def _edge_body_alias(ea_ref, nks_ref, nqd_ref, nvs_ref, weq_ref, beq_ref,
                     weo_ref, beo_ref, awf_ref, gs8_ref, k8_ref, ewbd_ref,
                     g1e_ref, b1e_ref, ehi_ref,
                     es8_ref, w_ref, eh_ref):
    del ehi_ref
    _edge_body(ea_ref, nks_ref, nqd_ref, nvs_ref, weq_ref, beq_ref,
               weo_ref, beo_ref, awf_ref, gs8_ref, k8_ref, ewbd_ref,
               g1e_ref, b1e_ref, es8_ref, w_ref, eh_ref)


def _edge_stage(ea, nks, nqd, nvs, weq, beq, weo, beo, awf, gs8, k8, ewbd,
                g1e, b1e, off, eh_init=None):
    ne = ea.shape[0]
    offb = off // BE
    grid = (ne // BE,)
    eblk = pl.BlockSpec((BE, D), lambda i: (i, 0))
    ehblk = pl.BlockSpec((BE, D), lambda i: (i + offb, 0))
    wblk = pl.BlockSpec((D, D), lambda i: (0, 0))
    vblk = pl.BlockSpec((1, D), lambda i: (0, 0))
    in_specs = [eblk, eblk, eblk, eblk, wblk, vblk, wblk, vblk, vblk,
                pl.BlockSpec((D, 8), lambda i: (0, 0)),
                pl.BlockSpec((8, D), lambda i: (0, 0)),
                wblk, vblk, vblk]
    args = [ea, nks, nqd, nvs, weq, beq, weo, beo, awf, gs8, k8, ewbd,
            g1e, b1e]
    body = _edge_body
    kw = {}
    if eh_init is not None:
        in_specs = in_specs + [pl.BlockSpec(memory_space=pl.ANY)]
        args = args + [eh_init]
        body = _edge_body_alias
        kw = dict(input_output_aliases={14: 2})
    return pl.pallas_call(
        body,
        grid=grid,
        in_specs=in_specs,
        out_specs=[pl.BlockSpec((BE, 8), lambda i: (i, 0)), eblk, ehblk],
        out_shape=[
            jax.ShapeDtypeStruct((ne, 8), _f32),
            jax.ShapeDtypeStruct((ne, D), _f32),
            jax.ShapeDtypeStruct((E, D), _f32),
        ],
        **kw,
    )(*args)


def _scatter_w_body(dst2_hbm, w_hbm, z_hbm, agg_hbm,
                    idx_raw, idx_loc, row0, row1, acc, s0, s1):
    cid = lax.axis_index("c")
    sid = lax.axis_index("s")
    r0 = sid * NROW_W
    pltpu.sync_copy(z_hbm.at[pl.ds(r0, NROW_W)], acc.at[pl.ds(r0, NROW_W)])

    pltpu.sync_copy(dst2_hbm.at[sid], idx_raw)
    base = cid * NHALF
    iota = lax.iota(jnp.int32, 16)

    def tbody(j, carry):
        jv = jnp.broadcast_to(j, (16,))
        for k in range(CH // 16):
            cv = k * 16 + iota
            raw = plsc.load_gather(idx_raw, [jv, cv])
            loc = raw - base
            inb = (loc >= 0) & (loc < NHALF)
            loc = jnp.where(inb, loc, NHALF)
            plsc.store_scatter(idx_loc, [jv, cv], loc)
        return carry

    lax.fori_loop(0, NCH_S, tbody, 0)
    plsc.subcore_barrier()

    e00 = sid * NCH_S * CH

    def fetch(j, row, sem):
        return pltpu.async_copy(w_hbm.at[pl.ds(e00 + j * CH, CH)], row, sem)

    def wait(j, row, sem):
        pltpu.make_async_copy(w_hbm.at[pl.ds(e00 + j * CH, CH)], row,
                              sem).wait()

    fetch(0, row0, s0)

    def body(i, carry):
        j = 2 * i
        wait(j, row0, s0)
        fetch(j + 1, row1, s1)
        pltpu.sync_copy(row0, acc.at[idx_loc.at[j]], add=True)
        wait(j + 1, row1, s1)
        fetch(j + 2, row0, s0)
        pltpu.sync_copy(row1, acc.at[idx_loc.at[j + 1]], add=True)
        return carry

    lax.fori_loop(0, NCH_S // 2, body, 0)
    j = NCH_S - 1
    wait(j, row0, s0)
    pltpu.sync_copy(row0, acc.at[idx_loc.at[j]], add=True)
    plsc.subcore_barrier()

    out0 = cid * ACC_ROWS + r0
    pltpu.sync_copy(acc.at[pl.ds(r0, NROW_W)], agg_hbm.at[pl.ds(out0, NROW_W)])


def _sc_scatter_w(dst2, w, z):
    mesh = plsc.VectorSubcoreMesh(core_axis_name="c", subcore_axis_name="s")
    f = functools.partial(
        pl.kernel,
        out_type=jax.ShapeDtypeStruct((2 * ACC_ROWS, D), _f32),
        mesh=mesh,
        compiler_params=pltpu.CompilerParams(needs_layout_passes=False),
        scratch_types=[
            pltpu.VMEM((NCH_S, CH), jnp.int32),
            pltpu.VMEM((NCH_S, CH), jnp.int32),
            pltpu.VMEM((CH, D), _f32),
            pltpu.VMEM((CH, D), _f32),
            pltpu.VMEM_SHARED((ACC_ROWS, D), _f32),
            pltpu.SemaphoreType.DMA,
            pltpu.SemaphoreType.DMA,
        ],
    )(_scatter_w_body)
    return f(dst2, w, z)


def _scatter_es_body(dst_hbm, es_hbm, z_hbm, ident_hbm, out_hbm,
                     idx_v, es_v, ident_v, acc, acc_sh, s0):
    cid = lax.axis_index("c")
    sid = lax.axis_index("s")
    wid = sid * NC + cid
    e0 = wid * EW
    pltpu.sync_copy(dst_hbm.at[pl.ds(e0, EW)], idx_v)
    pltpu.sync_copy(z_hbm.at[pl.ds(0, ES_ROWS)], acc)
    pltpu.sync_copy(z_hbm.at[pl.ds(sid * ES_SLAB, ES_SLAB)],
                    acc_sh.at[pl.ds(sid * ES_SLAB, ES_SLAB)])
    pltpu.sync_copy(ident_hbm, ident_v)

    iota = lax.iota(jnp.int32, 16)
    pair01 = jnp.where(iota >= 8, 1, 0)
    head = iota & 7

    def cbody(j, carry):
        pltpu.sync_copy(es_hbm.at[pl.ds((e0 + j * CH) * H, CH * H)], es_v)

        def pbody(m, c2):
            d2 = plsc.load_gather(idx_v, [j * CH + 2 * m + pair01])
            vals = plsc.load_gather(es_v, [16 * m + iota])
            sidx = d2 * H + head
            plsc.addupdate_scatter(acc, [sidx >> 7, sidx & 127], vals)
            return c2

        lax.fori_loop(0, CH // 2, pbody, 0)
        return carry

    lax.fori_loop(0, NCH_E, cbody, 0)
    plsc.subcore_barrier()
    for j in range(ES_ROWS // CH):
        pltpu.sync_copy(acc.at[pl.ds(j * CH, CH)],
                        acc_sh.at[ident_v.at[j]], add=True)
    plsc.subcore_barrier()
    pltpu.sync_copy(acc_sh.at[pl.ds(sid * ES_SLAB, ES_SLAB)],
                    out_hbm.at[pl.ds(cid * ES_ROWS + sid * ES_SLAB, ES_SLAB)])


def _sc_scatter_es(dst_flat, es_flat, z, ident):
    mesh = plsc.VectorSubcoreMesh(core_axis_name="c", subcore_axis_name="s")
    f = functools.partial(
        pl.kernel,
        out_type=jax.ShapeDtypeStruct((NC * ES_ROWS, D), _f32),
        mesh=mesh,
        compiler_params=pltpu.CompilerParams(needs_layout_passes=False),
        scratch_types=[
            pltpu.VMEM((EW,), jnp.int32),
            pltpu.VMEM((CH * H,), _f32),
            pltpu.VMEM((ES_ROWS // CH, CH), jnp.int32),
            pltpu.VMEM((ES_ROWS, D), _f32),
            pltpu.VMEM_SHARED((ES_ROWS, D), _f32),
            pltpu.SemaphoreType.DMA,
        ],
    )(_scatter_es_body)
    return f(dst_flat, es_flat, z, ident)


BN = 2000


def _node_body(sp0_ref, sp1_ref, agga_ref, aggb_ref, x_ref, ld_ref,
               k8_ref, deg0_ref, deg1_ref, wno_ref, bno_ref,
               g1h_ref, b1h_ref, w1_ref, b1_ref, w2_ref, b2_ref,
               g2h_ref, b2h_ref, out_ref):
    ssum = sp0_ref[...] + sp1_ref[...]
    rs = 1.0 / (ssum + 1e-16)
    rs128 = jnp.dot(rs, k8_ref[...], preferred_element_type=_f32)
    nh0 = (agga_ref[...] + aggb_ref[...]) * rs128
    nh = nh0 * deg0_ref[...] + (nh0 * ld_ref[...]) * deg1_ref[...]
    nh = jnp.dot(nh, wno_ref[...], preferred_element_type=_f32) + bno_ref[...]
    nh = nh + x_ref[...]
    m = jnp.mean(nh, axis=-1, keepdims=True)
    v = jnp.mean((nh - m) * (nh - m), axis=-1, keepdims=True)
    nh = (nh - m) * lax.rsqrt(v + 1e-5) * g1h_ref[...] + b1h_ref[...]
    t = nh
    h = jnp.maximum(jnp.dot(nh, w1_ref[...], preferred_element_type=_f32)
                    + b1_ref[...], 0.0)
    h = jnp.dot(h, w2_ref[...], preferred_element_type=_f32) + b2_ref[...]
    nh = t + h
    m = jnp.mean(nh, axis=-1, keepdims=True)
    v = jnp.mean((nh - m) * (nh - m), axis=-1, keepdims=True)
    nh = (nh - m) * lax.rsqrt(v + 1e-5) * g2h_ref[...] + b2h_ref[...]
    out_ref[...] = jnp.maximum(nh, 0.0)


def _node_stage(sp0, sp1, agga, aggb, x, ld, k8, deg0, deg1,
                wno, bno, g1h, b1h, w1, b1, w2, b2, g2h, b2h):
    grid = (N // BN,)
    nblk = pl.BlockSpec((BN, D), lambda i: (i, 0))
    vblk = pl.BlockSpec((1, D), lambda i: (0, 0))
    wblk = pl.BlockSpec((D, D), lambda i: (0, 0))
    sblk = pl.BlockSpec((BN, 8), lambda i: (i, 0))
    return pl.pallas_call(
        _node_body,
        grid=grid,
        in_specs=[sblk, sblk,
                  nblk, nblk, nblk,
                  pl.BlockSpec((BN, 1), lambda i: (i, 0)),
                  pl.BlockSpec((8, D), lambda i: (0, 0)),
                  vblk, vblk, wblk, vblk, vblk, vblk,
                  pl.BlockSpec((D, 2 * D), lambda i: (0, 0)),
                  pl.BlockSpec((1, 2 * D), lambda i: (0, 0)),
                  pl.BlockSpec((2 * D, D), lambda i: (0, 0)),
                  vblk, vblk, vblk],
        out_specs=nblk,
        out_shape=jax.ShapeDtypeStruct((N, D), _f32),
    )(sp0, sp1, agga, aggb, x, ld, k8, deg0, deg1,
      wno, bno, g1h, b1h, w1, b1, w2, b2, g2h, b2h)


def kernel(x, edge_attr, log_deg, params, edge_index):
    p = params
    src_g = edge_index[0].reshape(NW, NCH_G, CHG)
    dst_g = edge_index[1].reshape(NW, NCH_G, CHG)
    dst_s1 = edge_index[1][:E2].reshape(NS, NCH_S, CH)
    dst_s2 = edge_index[1][E2:].reshape(NS, NCH_S, CH)
    dst_flat = edge_index[1]

    awf = jnp.transpose(p['Aw'][:, :, 0]).reshape(1, D)
    gs8 = jnp.kron(jnp.eye(H, dtype=_f32), jnp.ones((HD, 1), _f32))
    k8 = jnp.kron(jnp.eye(H, dtype=_f32), jnp.ones((1, HD), _f32))
    ewbd = jax.scipy.linalg.block_diag(*[p['Ew'][:, h, :] for h in range(H)])
    deg0 = p['deg_coef'][:, :, 0]
    deg1 = p['deg_coef'][:, :, 1]
    zw = jnp.zeros((ACC_ROWS, D), _f32)
    ident = jnp.arange(ES_ROWS, dtype=jnp.int32).reshape(ES_ROWS // CH, CH)

    bq = p['bq'].reshape(1, D)
    beq = p['bEq'].reshape(1, D)
    beo = p['bEo'].reshape(1, D)
    bno = p['bNo'].reshape(1, D)
    b1 = p['b1'].reshape(1, 2 * D)
    b2 = p['b2'].reshape(1, D)
    g1e = p['g1e'].reshape(1, D)
    b1e = p['b1e'].reshape(1, D)
    g1h = p['g1h'].reshape(1, D)
    b1h = p['b1h'].reshape(1, D)
    g2h = p['g2h'].reshape(1, D)
    b2h = p['b2h'].reshape(1, D)

    nq, nk, nv = _node_proj(x, p['Wq'], p['Wk'], p['Wv'], bq)
    nks, nqd, nvs = _sc_gather(nk, nq, nv, src_g, dst_g)
    es8_1, w_1, eh_p = _edge_stage(
        edge_attr[:E2], nks[:E2], nqd[:E2], nvs[:E2], p['WEq'], beq,
        p['WEo'], beo, awf, gs8, k8, ewbd, g1e, b1e, 0)
    es8_2, w_2, eh = _edge_stage(
        edge_attr[E2:], nks[E2:], nqd[E2:], nvs[E2:], p['WEq'], beq,
        p['WEo'], beo, awf, gs8, k8, ewbd, g1e, b1e, E2, eh_init=eh_p)
    aggo1 = _sc_scatter_w(dst_s1, w_1, zw)
    aggo2 = _sc_scatter_w(dst_s2, w_2, zw)
    es8 = jnp.concatenate([es8_1, es8_2], axis=0)
    esp = _sc_scatter_es(dst_flat, es8.reshape(-1), zw, ident)

    agga = jnp.concatenate(
        [aggo1[:NHALF], aggo1[ACC_ROWS:ACC_ROWS + NHALF]], axis=0)
    aggb = jnp.concatenate(
        [aggo2[:NHALF], aggo2[ACC_ROWS:ACC_ROWS + NHALF]], axis=0)
    sp = esp.reshape(NC, NPAD_ES, H)

    nh = _node_stage(sp[0], sp[1], agga, aggb, x, log_deg, k8, deg0, deg1,
                     p['WNo'], bno, g1h, b1h, p['W1'], b1, p['W2'], b2,
                     g2h, b2h)
    return nh, eh


# speedup vs baseline: 1.3765x; 1.3765x over previous
import functools

import jax
import jax.numpy as jnp
from jax import lax
from jax.experimental import pallas as pl
from jax.experimental.pallas import tpu as pltpu
from jax.experimental.pallas import tpu_sc as plsc

N = 10000
E = 320000
D = 128
H = 8
HD = 16

NC = 2
NS = 16
NW = NC * NS

CH = 80
EW = E // NW
CHG = 80
NCH_G = EW // CHG
NP_G = NCH_G // 2
ET = E // NS
NCH_S = ET // CH
NCH_E = EW // CH

NHALF = N // 2
ACC_ROWS = 5120
NROW_W = ACC_ROWS // NS

NPAD_ES = 10240
ES_ROWS = NPAD_ES * H // D
ES_SLAB = ES_ROWS // NS

_f32 = jnp.float32


def _proj_body(x_ref, wq_ref, wk_ref, wv_ref, bq_ref, nq_ref, nk_ref, nv_ref):
    x = x_ref[...]
    nq_ref[...] = jnp.dot(x, wq_ref[...], preferred_element_type=_f32) + bq_ref[...]
    nk_ref[...] = jnp.dot(x, wk_ref[...], preferred_element_type=_f32)
    nv_ref[...] = jnp.dot(x, wv_ref[...], preferred_element_type=_f32)


BP = 2000


def _node_proj(x, wq, wk, wv, bq):
    nblk = pl.BlockSpec((BP, D), lambda i: (i, 0))
    wblk = pl.BlockSpec((D, D), lambda i: (0, 0))
    vblk = pl.BlockSpec((1, D), lambda i: (0, 0))
    return pl.pallas_call(
        _proj_body,
        grid=(N // BP,),
        in_specs=[nblk, wblk, wblk, wblk, vblk],
        out_specs=[nblk] * 3,
        out_shape=[jax.ShapeDtypeStruct((N, D), _f32)] * 3,
    )(x, wq, wk, wv, bq)


def _gather_body(nk_hbm, nq_hbm, nv_hbm, src2_hbm, dst2_hbm,
                 nks_hbm, nqd_hbm, nvs_hbm,
                 idxs_v, idxd_v, ra0, rb0, rc0, ra1, rb1, rc1,
                 sa0, sb0, sc0, sa1, sb1, sc1):
    wid = lax.axis_index("s") * NC + lax.axis_index("c")
    cb = wid * NCH_G
    pltpu.sync_copy(src2_hbm.at[wid], idxs_v)
    pltpu.sync_copy(dst2_hbm.at[wid], idxd_v)

    def issue(j, ra, rb, rc, sa, sb, sc_):
        da = pltpu.async_copy(nk_hbm.at[idxs_v.at[j]], ra, sa)
        db = pltpu.async_copy(nq_hbm.at[idxd_v.at[j]], rb, sb)
        dc = pltpu.async_copy(nv_hbm.at[idxs_v.at[j]], rc, sc_)
        return da, db, dc

    def store(j, ra, rb, rc):
        eb = (cb + j) * CHG
        pltpu.sync_copy(ra, nks_hbm.at[pl.ds(eb, CHG)])
        pltpu.sync_copy(rb, nqd_hbm.at[pl.ds(eb, CHG)])
        pltpu.sync_copy(rc, nvs_hbm.at[pl.ds(eb, CHG)])

    def wait(j, ra, rb, rc, sa, sb, sc_):
        pltpu.make_async_copy(nk_hbm.at[idxs_v.at[j]], ra, sa).wait()
        pltpu.make_async_copy(nq_hbm.at[idxd_v.at[j]], rb, sb).wait()
        pltpu.make_async_copy(nv_hbm.at[idxs_v.at[j]], rc, sc_).wait()

    issue(0, ra0, rb0, rc0, sa0, sb0, sc0)

    def body(i, carry):
        j = 2 * i
        wait(j, ra0, rb0, rc0, sa0, sb0, sc0)
        issue(j + 1, ra1, rb1, rc1, sa1, sb1, sc1)
        store(j, ra0, rb0, rc0)
        wait(j + 1, ra1, rb1, rc1, sa1, sb1, sc1)
        issue(j + 2, ra0, rb0, rc0, sa0, sb0, sc0)
        store(j + 1, ra1, rb1, rc1)
        return carry

    lax.fori_loop(0, NP_G, body, 0)
    j = NCH_G - 1
    wait(j, ra0, rb0, rc0, sa0, sb0, sc0)
    store(j, ra0, rb0, rc0)


def _sc_gather(nk, nq, nv, src2, dst2):
    mesh = plsc.VectorSubcoreMesh(core_axis_name="c", subcore_axis_name="s")
    rt = pltpu.VMEM((CHG, D), _f32)
    f = functools.partial(
        pl.kernel,
        out_type=[jax.ShapeDtypeStruct((E, D), _f32)] * 3,
        mesh=mesh,
        scratch_types=[
            pltpu.VMEM((NCH_G, CHG), jnp.int32),
            pltpu.VMEM((NCH_G, CHG), jnp.int32),
            rt, rt, rt, rt, rt, rt,
            pltpu.SemaphoreType.DMA,
            pltpu.SemaphoreType.DMA,
            pltpu.SemaphoreType.DMA,
            pltpu.SemaphoreType.DMA,
            pltpu.SemaphoreType.DMA,
            pltpu.SemaphoreType.DMA,
        ],
    )(_gather_body)
    return f(nk, nq, nv, src2, dst2)


BE = 2000


def _edge_body(ea_ref, nks_ref, nqd_ref, nvs_ref, weq_ref, beq_ref,
               weo_ref, beo_ref, awf_ref, gs8_ref, k8_ref, ewbd_ref,
               g1e_ref, b1e_ref,
               es8_ref, w_ref, eh_ref):
    ea = ea_ref[...]
    eq = jnp.dot(ea, weq_ref[...], preferred_element_type=_f32) + beq_ref[...]
    conn = jnp.maximum(nks_ref[...] + nqd_ref[...] + eq, 0.0)
    score8 = jnp.dot(conn * awf_ref[...], gs8_ref[...],
                     preferred_element_type=_f32)
    es8 = jnp.exp(jnp.clip(score8, -5.0, 5.0))
    es8_ref[...] = es8
    es128 = jnp.dot(es8, k8_ref[...], preferred_element_type=_f32)
    w_ref[...] = (nvs_ref[...]
                  + jnp.dot(conn, ewbd_ref[...], preferred_element_type=_f32)
                  ) * es128
    eh = ea + jnp.dot(conn, weo_ref[...], preferred_element_type=_f32) + beo_ref[...]
    m = jnp.mean(eh, axis=-1, keepdims=True)
    v = jnp.mean((eh - m) * (eh - m), axis=-1, keepdims=True)
    eh = (eh - m) * lax.rsqrt(v + 1e-5) * g1e_ref[...] + b1e_ref[...]
    eh_ref[...] = jnp.maximum(eh, 0.0)


def _edge_stage(ea, nks, nqd, nvs, weq, beq, weo, beo, awf, gs8, k8, ewbd,
                g1e, b1e):
    grid = (E // BE,)
    eblk = pl.BlockSpec((BE, D), lambda i: (i, 0))
    wblk = pl.BlockSpec((D, D), lambda i: (0, 0))
    vblk = pl.BlockSpec((1, D), lambda i: (0, 0))
    return pl.pallas_call(
        _edge_body,
        grid=grid,
        in_specs=[eblk, eblk, eblk, eblk, wblk, vblk, wblk, vblk, vblk,
                  pl.BlockSpec((D, 8), lambda i: (0, 0)),
                  pl.BlockSpec((8, D), lambda i: (0, 0)),
                  wblk, vblk, vblk],
        out_specs=[pl.BlockSpec((BE, 8), lambda i: (i, 0)), eblk, eblk],
        out_shape=[
            jax.ShapeDtypeStruct((E, 8), _f32),
            jax.ShapeDtypeStruct((E, D), _f32),
            jax.ShapeDtypeStruct((E, D), _f32),
        ],
    )(ea, nks, nqd, nvs, weq, beq, weo, beo, awf, gs8, k8, ewbd, g1e, b1e)


def _scatter_w_body(dst2_hbm, w_hbm, z_hbm, agg_hbm,
                    idx_raw, idx_loc, row0, row1, acc, s0, s1):
    cid = lax.axis_index("c")
    sid = lax.axis_index("s")
    r0 = sid * NROW_W
    pltpu.sync_copy(z_hbm.at[pl.ds(r0, NROW_W)], acc.at[pl.ds(r0, NROW_W)])

    pltpu.sync_copy(dst2_hbm.at[sid], idx_raw)
    base = cid * NHALF
    iota = lax.iota(jnp.int32, 16)

    def tbody(j, carry):
        jv = jnp.broadcast_to(j, (16,))
        for k in range(CH // 16):
            cv = k * 16 + iota
            raw = plsc.load_gather(idx_raw, [jv, cv])
            loc = raw - base
            inb = (loc >= 0) & (loc < NHALF)
            loc = jnp.where(inb, loc, NHALF)
            plsc.store_scatter(idx_loc, [jv, cv], loc)
        return carry

    lax.fori_loop(0, NCH_S, tbody, 0)
    plsc.subcore_barrier()

    e00 = sid * NCH_S * CH

    def fetch(j, row, sem):
        return pltpu.async_copy(w_hbm.at[pl.ds(e00 + j * CH, CH)], row, sem)

    def wait(j, row, sem):
        pltpu.make_async_copy(w_hbm.at[pl.ds(e00 + j * CH, CH)], row,
                              sem).wait()

    fetch(0, row0, s0)

    def body(i, carry):
        j = 2 * i
        wait(j, row0, s0)
        fetch(j + 1, row1, s1)
        pltpu.sync_copy(row0, acc.at[idx_loc.at[j]], add=True)
        wait(j + 1, row1, s1)
        fetch(j + 2, row0, s0)
        pltpu.sync_copy(row1, acc.at[idx_loc.at[j + 1]], add=True)
        return carry

    lax.fori_loop(0, NCH_S // 2 - 1, body, 0)
    j = NCH_S - 2
    wait(j, row0, s0)
    fetch(j + 1, row1, s1)
    pltpu.sync_copy(row0, acc.at[idx_loc.at[j]], add=True)
    wait(j + 1, row1, s1)
    pltpu.sync_copy(row1, acc.at[idx_loc.at[j + 1]], add=True)
    plsc.subcore_barrier()

    out0 = cid * ACC_ROWS + r0
    pltpu.sync_copy(acc.at[pl.ds(r0, NROW_W)], agg_hbm.at[pl.ds(out0, NROW_W)])


def _sc_scatter_w(dst2, w, z):
    mesh = plsc.VectorSubcoreMesh(core_axis_name="c", subcore_axis_name="s")
    f = functools.partial(
        pl.kernel,
        out_type=jax.ShapeDtypeStruct((2 * ACC_ROWS, D), _f32),
        mesh=mesh,
        compiler_params=pltpu.CompilerParams(needs_layout_passes=False),
        scratch_types=[
            pltpu.VMEM((NCH_S, CH), jnp.int32),
            pltpu.VMEM((NCH_S, CH), jnp.int32),
            pltpu.VMEM((CH, D), _f32),
            pltpu.VMEM((CH, D), _f32),
            pltpu.VMEM_SHARED((ACC_ROWS, D), _f32),
            pltpu.SemaphoreType.DMA,
            pltpu.SemaphoreType.DMA,
        ],
    )(_scatter_w_body)
    return f(dst2, w, z)


def _scatter_es_body(dst_hbm, es_hbm, z_hbm, ident_hbm, out_hbm,
                     idx_v, es_v0, es_v1, ident_v, acc, acc_sh, s0, s1):
    cid = lax.axis_index("c")
    sid = lax.axis_index("s")
    wid = sid * NC + cid
    e0 = wid * EW
    pltpu.sync_copy(dst_hbm.at[pl.ds(e0, EW)], idx_v)
    pltpu.sync_copy(z_hbm.at[pl.ds(0, ES_ROWS)], acc)
    pltpu.sync_copy(z_hbm.at[pl.ds(sid * ES_SLAB, ES_SLAB)],
                    acc_sh.at[pl.ds(sid * ES_SLAB, ES_SLAB)])
    pltpu.sync_copy(ident_hbm, ident_v)

    iota = lax.iota(jnp.int32, 16)
    pair01 = jnp.where(iota >= 8, 1, 0)
    head = iota & 7

    def fetch(j, buf, sem):
        pltpu.async_copy(es_hbm.at[pl.ds((e0 + j * CH) * H, CH * H)],
                         buf, sem)

    def waitf(j, buf, sem):
        pltpu.make_async_copy(es_hbm.at[pl.ds((e0 + j * CH) * H, CH * H)],
                              buf, sem).wait()

    def work(j, buf):
        def pbody(m, c2):
            d2 = plsc.load_gather(idx_v, [j * CH + 2 * m + pair01])
            vals = plsc.load_gather(buf, [16 * m + iota])
            sidx = d2 * H + head
            plsc.addupdate_scatter(acc, [sidx >> 7, sidx & 127], vals)
            return c2

        lax.fori_loop(0, CH // 2, pbody, 0)

    fetch(0, es_v0, s0)

    def cbody(i, carry):
        j = 2 * i
        waitf(j, es_v0, s0)
        fetch(j + 1, es_v1, s1)
        work(j, es_v0)
        waitf(j + 1, es_v1, s1)
        fetch(j + 2, es_v0, s0)
        work(j + 1, es_v1)
        return carry

    lax.fori_loop(0, NCH_E // 2, cbody, 0)
    j = NCH_E - 1
    waitf(j, es_v0, s0)
    work(j, es_v0)
    plsc.subcore_barrier()
    for j in range(ES_ROWS // CH):
        pltpu.sync_copy(acc.at[pl.ds(j * CH, CH)],
                        acc_sh.at[ident_v.at[j]], add=True)
    plsc.subcore_barrier()
    pltpu.sync_copy(acc_sh.at[pl.ds(sid * ES_SLAB, ES_SLAB)],
                    out_hbm.at[pl.ds(cid * ES_ROWS + sid * ES_SLAB, ES_SLAB)])


def _sc_scatter_es(dst_flat, es_flat, z, ident):
    mesh = plsc.VectorSubcoreMesh(core_axis_name="c", subcore_axis_name="s")
    f = functools.partial(
        pl.kernel,
        out_type=jax.ShapeDtypeStruct((NC * ES_ROWS, D), _f32),
        mesh=mesh,
        compiler_params=pltpu.CompilerParams(needs_layout_passes=False),
        scratch_types=[
            pltpu.VMEM((EW,), jnp.int32),
            pltpu.VMEM((CH * H,), _f32),
            pltpu.VMEM((CH * H,), _f32),
            pltpu.VMEM((ES_ROWS // CH, CH), jnp.int32),
            pltpu.VMEM((ES_ROWS, D), _f32),
            pltpu.VMEM_SHARED((ES_ROWS, D), _f32),
            pltpu.SemaphoreType.DMA,
            pltpu.SemaphoreType.DMA,
        ],
    )(_scatter_es_body)
    return f(dst_flat, es_flat, z, ident)


BN = 2000


def _node_body(sp0_ref, sp1_ref, agg_ref, x_ref, ld_ref,
               k8_ref, deg0_ref, deg1_ref, wno_ref, bno_ref,
               g1h_ref, b1h_ref, w1_ref, b1_ref, w2_ref, b2_ref,
               g2h_ref, b2h_ref, out_ref):
    ssum = sp0_ref[...] + sp1_ref[...]
    rs = 1.0 / (ssum + 1e-16)
    rs128 = jnp.dot(rs, k8_ref[...], preferred_element_type=_f32)
    nh0 = agg_ref[...] * rs128
    nh = nh0 * deg0_ref[...] + (nh0 * ld_ref[...]) * deg1_ref[...]
    nh = jnp.dot(nh, wno_ref[...], preferred_element_type=_f32) + bno_ref[...]
    nh = nh + x_ref[...]
    m = jnp.mean(nh, axis=-1, keepdims=True)
    v = jnp.mean((nh - m) * (nh - m), axis=-1, keepdims=True)
    nh = (nh - m) * lax.rsqrt(v + 1e-5) * g1h_ref[...] + b1h_ref[...]
    t = nh
    h = jnp.maximum(jnp.dot(nh, w1_ref[...], preferred_element_type=_f32)
                    + b1_ref[...], 0.0)
    h = jnp.dot(h, w2_ref[...], preferred_element_type=_f32) + b2_ref[...]
    nh = t + h
    m = jnp.mean(nh, axis=-1, keepdims=True)
    v = jnp.mean((nh - m) * (nh - m), axis=-1, keepdims=True)
    nh = (nh - m) * lax.rsqrt(v + 1e-5) * g2h_ref[...] + b2h_ref[...]
    out_ref[...] = jnp.maximum(nh, 0.0)


def _node_stage(sp0, sp1, agg, x, ld, k8, deg0, deg1,
                wno, bno, g1h, b1h, w1, b1, w2, b2, g2h, b2h):
    grid = (N // BN,)
    nblk = pl.BlockSpec((BN, D), lambda i: (i, 0))
    vblk = pl.BlockSpec((1, D), lambda i: (0, 0))
    wblk = pl.BlockSpec((D, D), lambda i: (0, 0))
    sblk = pl.BlockSpec((BN, 8), lambda i: (i, 0))
    return pl.pallas_call(
        _node_body,
        grid=grid,
        in_specs=[sblk, sblk,
                  nblk, nblk,
                  pl.BlockSpec((BN, 1), lambda i: (i, 0)),
                  pl.BlockSpec((8, D), lambda i: (0, 0)),
                  vblk, vblk, wblk, vblk, vblk, vblk,
                  pl.BlockSpec((D, 2 * D), lambda i: (0, 0)),
                  pl.BlockSpec((1, 2 * D), lambda i: (0, 0)),
                  pl.BlockSpec((2 * D, D), lambda i: (0, 0)),
                  vblk, vblk, vblk],
        out_specs=nblk,
        out_shape=jax.ShapeDtypeStruct((N, D), _f32),
    )(sp0, sp1, agg, x, ld, k8, deg0, deg1,
      wno, bno, g1h, b1h, w1, b1, w2, b2, g2h, b2h)


def kernel(x, edge_attr, log_deg, params, edge_index):
    p = params
    src_g = edge_index[0].reshape(NW, NCH_G, CHG)
    dst_g = edge_index[1].reshape(NW, NCH_G, CHG)
    dst_s = edge_index[1].reshape(NS, NCH_S, CH)
    dst_flat = edge_index[1]

    awf = jnp.transpose(p['Aw'][:, :, 0]).reshape(1, D)
    gs8 = jnp.kron(jnp.eye(H, dtype=_f32), jnp.ones((HD, 1), _f32))
    k8 = jnp.kron(jnp.eye(H, dtype=_f32), jnp.ones((1, HD), _f32))
    ewbd = jax.scipy.linalg.block_diag(*[p['Ew'][:, h, :] for h in range(H)])
    deg0 = p['deg_coef'][:, :, 0]
    deg1 = p['deg_coef'][:, :, 1]
    zw = jnp.zeros((ACC_ROWS, D), _f32)
    ident = jnp.arange(ES_ROWS, dtype=jnp.int32).reshape(ES_ROWS // CH, CH)

    bq = p['bq'].reshape(1, D)
    beq = p['bEq'].reshape(1, D)
    beo = p['bEo'].reshape(1, D)
    bno = p['bNo'].reshape(1, D)
    b1 = p['b1'].reshape(1, 2 * D)
    b2 = p['b2'].reshape(1, D)
    g1e = p['g1e'].reshape(1, D)
    b1e = p['b1e'].reshape(1, D)
    g1h = p['g1h'].reshape(1, D)
    b1h = p['b1h'].reshape(1, D)
    g2h = p['g2h'].reshape(1, D)
    b2h = p['b2h'].reshape(1, D)

    nq, nk, nv = _node_proj(x, p['Wq'], p['Wk'], p['Wv'], bq)
    nks, nqd, nvs = _sc_gather(nk, nq, nv, src_g, dst_g)
    es8, w, eh = _edge_stage(edge_attr, nks, nqd, nvs, p['WEq'], beq,
                             p['WEo'], beo, awf, gs8, k8, ewbd, g1e, b1e)
    aggo = _sc_scatter_w(dst_s, w, zw)
    esp = _sc_scatter_es(dst_flat, es8.reshape(-1), zw, ident)

    agg = jnp.concatenate(
        [aggo[:NHALF], aggo[ACC_ROWS:ACC_ROWS + NHALF]], axis=0)
    sp = esp.reshape(NC, NPAD_ES, H)

    nh = _node_stage(sp[0], sp[1], agg, x, log_deg, k8, deg0, deg1,
                     p['WNo'], bno, g1h, b1h, p['W1'], b1, p['W2'], b2,
                     g2h, b2h)
    return nh, eh
